# Initial kernel scaffold; baseline (speedup 1.0000x reference)
import jax
import jax.numpy as jnp
from jax.experimental import pallas as pl


def kernel(atomics, pos, edge_index, edge_attr, batch, emb1, pos_W, pos_b, mlp_W1, mlp_b1, mlp_W2, mlp_b2, edge_emb, bn_g, bn_b, feat_W, feat_b, head_W1, head_b1, head_W2, head_b2):
    raise NotImplementedError("write your pallas kernel here")


# speedup vs baseline: 1.5502x; 1.5502x over previous
import functools
import jax
import jax.numpy as jnp
from jax import lax
from jax.experimental import pallas as pl
from jax.experimental.pallas import tpu as pltpu, tpu_sc as plsc

N_NODES = 10000
N_EDGES = 320000
EMB_DIM = 128
FEAT_DIM = 256
NUM_LAYERS = 5
NUM_GRAPHS = 256
EPS = 1e-5

NC = 2
NS = 16
NW = NC * NS
CH = 128
GCH = -(-N_EDGES // (NW * CH))
E_PAD = NW * CH * GCH
TILE_ROWS = 632
ACC_ROWS = TILE_ROWS * NS
DUMMY_ROW = N_NODES
ROWS_OUT = 624
TAIL_OFF = ROWS_OUT * NS
TAIL_ROWS = N_NODES - TAIL_OFF

ROW_BLK = 2000
NBLK = N_NODES // ROW_BLK


@functools.lru_cache(maxsize=None)
def _make_sc_scatter(d):
  mesh = plsc.VectorSubcoreMesh(core_axis_name="c", subcore_axis_name="s",
                                num_cores=NC, num_subcores=NS)

  @functools.partial(
      pl.kernel,
      out_type=jax.ShapeDtypeStruct((NC, N_NODES, d), jnp.float32),
      mesh=mesh,
      scratch_types=[
          pltpu.VMEM((GCH, CH), jnp.int32),
          pltpu.VMEM((GCH, CH), jnp.int32),
          pltpu.VMEM((CH, d), jnp.float32),
          pltpu.VMEM_SHARED((ACC_ROWS, d), jnp.float32),
          pltpu.SemaphoreType.DMA,
      ],
  )
  def k(table_hbm, srcg_hbm, dstg_hbm, zeros_hbm, out_hbm,
        src_v, dst_v, rows_v, acc_sh, sem):
    c = lax.axis_index("c")
    s = lax.axis_index("s")
    wid = c * NS + s
    pltpu.sync_copy(srcg_hbm.at[wid], src_v)
    pltpu.sync_copy(dstg_hbm.at[wid], dst_v)
    pltpu.sync_copy(zeros_hbm, acc_sh.at[pl.ds(s * TILE_ROWS, TILE_ROWS)])
    plsc.subcore_barrier()

    def body(j, carry):
      pltpu.async_copy(table_hbm.at[src_v.at[j]], rows_v, sem).wait()
      pltpu.sync_copy(rows_v, acc_sh.at[dst_v.at[j]], add=True)
      return carry

    lax.fori_loop(0, GCH, body, 0)
    plsc.subcore_barrier()
    pltpu.sync_copy(acc_sh.at[pl.ds(s * ROWS_OUT, ROWS_OUT)],
                    out_hbm.at[c, pl.ds(s * ROWS_OUT, ROWS_OUT)])

    @pl.when(s == NS - 1)
    def _():
      pltpu.sync_copy(acc_sh.at[pl.ds(TAIL_OFF, TAIL_ROWS)],
                      out_hbm.at[c, pl.ds(TAIL_OFF, TAIL_ROWS)])

  return k


def _sc_scatter_feat(*args):
  return _make_sc_scatter(EMB_DIM)(*args)


def _sc_scatter_cnt(*args):
  return _make_sc_scatter(EMB_DIM)(*args)


def _softplus(x):
  return jnp.maximum(x, 0.0) + jnp.log1p(jnp.exp(-jnp.abs(x)))


def _embed_body(atom_ref, pos_ref, emb_ref, posw_ref, posb_ref, out_ref):
  oh = (atom_ref[...] == lax.broadcasted_iota(jnp.int32, (ROW_BLK, 128), 1))
  oh = oh.astype(jnp.float32)
  out_ref[...] = (jnp.dot(oh, emb_ref[...], preferred_element_type=jnp.float32)
                  + jnp.dot(pos_ref[...], posw_ref[...],
                            preferred_element_type=jnp.float32)
                  + posb_ref[...])


def _embed(atomics2d, pos8, emb1p, posw8, posb):
  return pl.pallas_call(
      _embed_body,
      grid=(NBLK,),
      in_specs=[
          pl.BlockSpec((ROW_BLK, 1), lambda i: (i, 0)),
          pl.BlockSpec((ROW_BLK, 8), lambda i: (i, 0)),
          pl.BlockSpec((128, 128), lambda i: (0, 0)),
          pl.BlockSpec((8, 128), lambda i: (0, 0)),
          pl.BlockSpec((1, 128), lambda i: (0, 0)),
      ],
      out_specs=pl.BlockSpec((ROW_BLK, 128), lambda i: (i, 0)),
      out_shape=jax.ShapeDtypeStruct((N_NODES, EMB_DIM), jnp.float32),
  )(atomics2d, pos8, emb1p, posw8, posb)


def _layer_mm_body(parts_ref, h_ref, cnt_ref, ee_ref, w1_ref, b1_ref,
                   w2_ref, b2_ref, h2_ref, mom_ref, acc_ref):
  i = pl.program_id(0)
  cnt = cnt_ref[0] + cnt_ref[1]
  ctot = cnt[:, 0:1]
  c1 = cnt[:, 1:2]
  e0 = ee_ref[0:1, :]
  e1 = ee_ref[1:2, :]
  agg = (parts_ref[0] + parts_ref[1] + h_ref[...]
         + ctot * e0 + c1 * (e1 - e0) + e1)
  t = _softplus(jnp.dot(agg, w1_ref[...], preferred_element_type=jnp.float32)
                + b1_ref[...])
  h2 = jnp.dot(t, w2_ref[...], preferred_element_type=jnp.float32) + b2_ref[...]
  h2_ref[...] = h2

  @pl.when(i == 0)
  def _():
    acc_ref[...] = jnp.zeros_like(acc_ref)

  acc_ref[0:1, :] += jnp.sum(h2, axis=0)[None, :]
  acc_ref[1:2, :] += jnp.sum(h2 * h2, axis=0)[None, :]

  @pl.when(i == NBLK - 1)
  def _():
    mom_ref[...] = acc_ref[...]


def _layer_mm(parts, h, cntp, ee, w1, b1, w2, b2):
  return pl.pallas_call(
      _layer_mm_body,
      grid=(NBLK,),
      in_specs=[
          pl.BlockSpec((2, ROW_BLK, 128), lambda i: (0, i, 0)),
          pl.BlockSpec((ROW_BLK, 128), lambda i: (i, 0)),
          pl.BlockSpec((2, ROW_BLK, 128), lambda i: (0, i, 0)),
          pl.BlockSpec((2, 128), lambda i: (0, 0)),
          pl.BlockSpec((128, 256), lambda i: (0, 0)),
          pl.BlockSpec((1, 256), lambda i: (0, 0)),
          pl.BlockSpec((256, 128), lambda i: (0, 0)),
          pl.BlockSpec((1, 128), lambda i: (0, 0)),
      ],
      out_specs=[
          pl.BlockSpec((ROW_BLK, 128), lambda i: (i, 0)),
          pl.BlockSpec((8, 128), lambda i: (0, 0)),
      ],
      out_shape=[
          jax.ShapeDtypeStruct((N_NODES, EMB_DIM), jnp.float32),
          jax.ShapeDtypeStruct((8, EMB_DIM), jnp.float32),
      ],
      scratch_shapes=[pltpu.VMEM((8, 128), jnp.float32)],
  )(parts, h, cntp, ee, w1, b1, w2, b2)


def _make_bn_body(last):
  def body(h2_ref, mom_ref, g_ref, b_ref, out_ref):
    n = jnp.float32(N_NODES)
    mean = mom_ref[0:1, :] / n
    var = mom_ref[1:2, :] / n - mean * mean
    rstd = lax.rsqrt(var + EPS)
    y = (h2_ref[...] - mean) * (rstd * g_ref[...]) + b_ref[...]
    if not last:
      y = _softplus(y)
    out_ref[...] = y
  return body


def _bn(h2, mom, g, b, last):
  return pl.pallas_call(
      _make_bn_body(last),
      grid=(NBLK,),
      in_specs=[
          pl.BlockSpec((ROW_BLK, 128), lambda i: (i, 0)),
          pl.BlockSpec((8, 128), lambda i: (0, 0)),
          pl.BlockSpec((1, 128), lambda i: (0, 0)),
          pl.BlockSpec((1, 128), lambda i: (0, 0)),
      ],
      out_specs=pl.BlockSpec((ROW_BLK, 128), lambda i: (i, 0)),
      out_shape=jax.ShapeDtypeStruct((N_NODES, EMB_DIM), jnp.float32),
  )(h2, mom, g, b)


def _pool_head_body(h_ref, batch_ref, fw_ref, fb_ref, hw1_ref, hb1_ref,
                    hw2_ref, hb2_ref, out_ref, pool_ref, cnt_ref):
  i = pl.program_id(0)

  @pl.when(i == 0)
  def _():
    pool_ref[...] = jnp.zeros_like(pool_ref)
    cnt_ref[...] = jnp.zeros_like(cnt_ref)

  feat = (jnp.dot(h_ref[...], fw_ref[...], preferred_element_type=jnp.float32)
          + fb_ref[...])
  oh = (batch_ref[...] == lax.broadcasted_iota(jnp.int32, (ROW_BLK, 256), 1))
  oh = oh.astype(jnp.float32)
  dn = (((0,), (0,)), ((), ()))
  pool_ref[...] += lax.dot_general(oh, feat, dimension_numbers=dn,
                                   preferred_element_type=jnp.float32)
  ones = jnp.ones((ROW_BLK, 128), jnp.float32)
  cnt_ref[...] += lax.dot_general(oh, ones, dimension_numbers=dn,
                                  preferred_element_type=jnp.float32)

  @pl.when(i == NBLK - 1)
  def _():
    counts = jnp.maximum(cnt_ref[:, 0:1], 1.0)
    pooled = pool_ref[...] / counts
    u = _softplus(jnp.dot(pooled, hw1_ref[...],
                          preferred_element_type=jnp.float32) + hb1_ref[...])
    out_ref[...] = (jnp.dot(u, hw2_ref[...],
                            preferred_element_type=jnp.float32) + hb2_ref[...])


def _pool_head(h, batch2d, fw, fb, hw1, hb1, hw2p, hb2p):
  return pl.pallas_call(
      _pool_head_body,
      grid=(NBLK,),
      in_specs=[
          pl.BlockSpec((ROW_BLK, 128), lambda i: (i, 0)),
          pl.BlockSpec((ROW_BLK, 1), lambda i: (i, 0)),
          pl.BlockSpec((128, 256), lambda i: (0, 0)),
          pl.BlockSpec((1, 256), lambda i: (0, 0)),
          pl.BlockSpec((256, 128), lambda i: (0, 0)),
          pl.BlockSpec((1, 128), lambda i: (0, 0)),
          pl.BlockSpec((128, 128), lambda i: (0, 0)),
          pl.BlockSpec((1, 128), lambda i: (0, 0)),
      ],
      out_specs=pl.BlockSpec((256, 128), lambda i: (0, 0)),
      out_shape=jax.ShapeDtypeStruct((NUM_GRAPHS, 128), jnp.float32),
      scratch_shapes=[
          pltpu.VMEM((256, 256), jnp.float32),
          pltpu.VMEM((256, 128), jnp.float32),
      ],
  )(h, batch2d, fw, fb, hw1, hb1, hw2p, hb2p)


def kernel(atomics, pos, edge_index, edge_attr, batch, emb1, pos_W, pos_b,
           mlp_W1, mlp_b1, mlp_W2, mlp_b2, edge_emb, bn_g, bn_b,
           feat_W, feat_b, head_W1, head_b1, head_W2, head_b2):
  f32 = jnp.float32
  i32 = jnp.int32

  npad = E_PAD - N_EDGES
  src = jnp.concatenate([edge_index[0].astype(i32),
                         jnp.zeros((npad,), i32)])
  dst = jnp.concatenate([edge_index[1].astype(i32),
                         jnp.full((npad,), DUMMY_ROW, i32)])
  ea = jnp.concatenate([edge_attr.astype(i32), jnp.zeros((npad,), i32)])
  srcg = src.reshape(NW, GCH, CH)
  dstg = dst.reshape(NW, GCH, CH)
  eag = ea.reshape(NW, GCH, CH)
  zeros_feat = jnp.zeros((TILE_ROWS, EMB_DIM), f32)
  cnt_table = jnp.zeros((2, EMB_DIM), f32).at[:, 0].set(1.0).at[1, 1].set(1.0)

  atomics2d = atomics.astype(i32).reshape(N_NODES, 1)
  batch2d = batch.astype(i32).reshape(N_NODES, 1)
  pos8 = jnp.pad(pos.astype(f32), ((0, 0), (0, 5)))
  emb1p = jnp.pad(emb1, ((0, 128 - emb1.shape[0]), (0, 0)))
  posw8 = jnp.pad(pos_W, ((0, 5), (0, 0)))
  posb = pos_b.reshape(1, EMB_DIM)
  hw2p = jnp.pad(head_W2, ((0, 0), (0, 127)))
  hb2p = jnp.pad(head_b2, ((0, 127),)).reshape(1, 128)

  cntp = _sc_scatter_cnt(cnt_table, eag, dstg, zeros_feat)

  h = _embed(atomics2d, pos8, emb1p, posw8, posb)

  for l in range(NUM_LAYERS):
    parts = _sc_scatter_feat(h, srcg, dstg, zeros_feat)
    h2, mom = _layer_mm(parts, h, cntp, edge_emb[l],
                        mlp_W1[l], mlp_b1[l].reshape(1, 2 * EMB_DIM),
                        mlp_W2[l], mlp_b2[l].reshape(1, EMB_DIM))
    h = _bn(h2, mom, bn_g[l].reshape(1, EMB_DIM), bn_b[l].reshape(1, EMB_DIM),
            last=(l == NUM_LAYERS - 1))

  out = _pool_head(h, batch2d, feat_W, feat_b.reshape(1, FEAT_DIM),
                   head_W1, head_b1.reshape(1, FEAT_DIM // 2), hw2p, hb2p)
  return out[:, :1]
